# baseline (speedup 1.0000x reference)
import functools

import jax
import jax.numpy as jnp
from jax import lax
from jax.experimental import pallas as pl
from jax.experimental.pallas import tpu as pltpu
from jax.experimental.pallas import tpu_sc as plsc

D_MODEL = 1024

NC = 2
NS = 16
NW = NC * NS

C = 16
NBUF = 4


@functools.lru_cache(maxsize=None)
def _build(n_chunks: int, d_model: int):
    b_per_w = n_chunks * C

    def body(tok_hbm, table_hbm, out_hbm, idx_v, buf_v, sems):
        wid = lax.axis_index("s") * NC + lax.axis_index("c")
        base = wid * b_per_w
        pltpu.sync_copy(tok_hbm.at[wid], idx_v)
        lead = NBUF - 1
        for j in range(lead):
            pltpu.async_copy(table_hbm.at[idx_v.at[j]], buf_v.at[j], sems.at[j])

        @pl.loop(0, n_chunks, step=NBUF)
        def _(c):
            for b in range(NBUF):
                cc = c + b
                pltpu.make_async_copy(
                    table_hbm.at[idx_v.at[cc]], buf_v.at[b], sems.at[b]
                ).wait()
                pltpu.sync_copy(
                    buf_v.at[b], out_hbm.at[pl.ds(base + cc * C, C)]
                )
                nxt = cc + lead
                nb = (b + lead) % NBUF

                @pl.when(nxt < n_chunks)
                def _():
                    pltpu.async_copy(
                        table_hbm.at[idx_v.at[nxt]], buf_v.at[nb], sems.at[nb]
                    )

    return pl.kernel(
        body,
        out_type=jax.ShapeDtypeStruct((NW * b_per_w, d_model), jnp.float32),
        mesh=plsc.VectorSubcoreMesh(core_axis_name="c", subcore_axis_name="s"),
        scratch_types=[
            pltpu.VMEM((n_chunks, C), jnp.int32),
            pltpu.VMEM((NBUF, C, d_model), jnp.float32),
            pltpu.SemaphoreType.DMA((NBUF,)),
        ],
    )


def kernel(tokens, embed_weights):
    n_tokens = tokens.size
    d_model = embed_weights.shape[1]
    grain = NW * C * NBUF
    n_pad = (-n_tokens) % grain
    tok_flat = tokens.reshape(-1).astype(jnp.int32)
    if n_pad:
        tok_flat = jnp.concatenate([tok_flat, jnp.zeros((n_pad,), jnp.int32)])
    n_chunks = tok_flat.size // (NW * C)
    tok3 = tok_flat.reshape(NW, n_chunks, C)
    out = _build(n_chunks, d_model)(tok3, embed_weights)
    if n_pad:
        out = out[:n_tokens]
    return out.reshape(tokens.shape + (d_model,))


# speedup vs baseline: 1.0092x; 1.0092x over previous
import functools

import jax
import jax.numpy as jnp
from jax import lax
from jax.experimental import pallas as pl
from jax.experimental.pallas import tpu as pltpu
from jax.experimental.pallas import tpu_sc as plsc

NC = 2
NS = 16
NW = NC * NS

C = 16
NBUF = 4


@functools.lru_cache(maxsize=None)
def _build(n_rows: int, n_cols: int, d_model: int):
    b_per_w = n_rows * n_cols // NW
    n_chunks = b_per_w // C
    w_per_row = n_cols // b_per_w

    def body(tok_hbm, table_hbm, out_hbm, idx_v, buf_v, sems):
        wid = lax.axis_index("s") * NC + lax.axis_index("c")
        base = wid * b_per_w
        pltpu.sync_copy(
            tok_hbm.at[wid // w_per_row,
                       pl.ds((wid % w_per_row) * b_per_w, b_per_w)],
            idx_v,
        )
        lead = NBUF - 1
        for j in range(lead):
            pltpu.async_copy(
                table_hbm.at[idx_v.at[pl.ds(j * C, C)]], buf_v.at[j], sems.at[j]
            )

        @pl.loop(0, n_chunks, step=NBUF)
        def _(c):
            for b in range(NBUF):
                cc = c + b
                pltpu.make_async_copy(
                    table_hbm.at[idx_v.at[pl.ds(cc * C, C)]], buf_v.at[b],
                    sems.at[b],
                ).wait()
                pltpu.sync_copy(
                    buf_v.at[b], out_hbm.at[pl.ds(base + cc * C, C)]
                )
                nxt = cc + lead
                nb = (b + lead) % NBUF

                @pl.when(nxt < n_chunks)
                def _():
                    pltpu.async_copy(
                        table_hbm.at[idx_v.at[pl.ds(nxt * C, C)]], buf_v.at[nb],
                        sems.at[nb],
                    )

    return pl.kernel(
        body,
        out_type=jax.ShapeDtypeStruct((NW * b_per_w, d_model), jnp.float32),
        mesh=plsc.VectorSubcoreMesh(core_axis_name="c", subcore_axis_name="s"),
        scratch_types=[
            pltpu.VMEM((b_per_w,), jnp.int32),
            pltpu.VMEM((NBUF, C, d_model), jnp.float32),
            pltpu.SemaphoreType.DMA((NBUF,)),
        ],
    )


def _lookup_flat(tok_flat, embed_weights):
    d_model = embed_weights.shape[1]
    grain = NW * C * NBUF
    n_pad = (-tok_flat.size) % grain
    n_tokens = tok_flat.size
    if n_pad:
        tok_flat = jnp.concatenate([tok_flat, jnp.zeros((n_pad,), jnp.int32)])
    out = _build(1, tok_flat.size, d_model)(tok_flat.reshape(1, -1),
                                            embed_weights)
    if n_pad:
        out = out[:n_tokens]
    return out


def kernel(tokens, embed_weights):
    d_model = embed_weights.shape[1]
    tok = tokens.astype(jnp.int32)
    n_rows, n_cols = (tok.shape if tok.ndim == 2 else (1, tok.size))
    b_per_w = n_rows * n_cols // NW
    if (tok.ndim == 2 and n_rows * n_cols % (NW * C * NBUF) == 0
            and b_per_w <= n_cols and n_cols % b_per_w == 0):
        out = _build(n_rows, n_cols, d_model)(tok, embed_weights)
    else:
        out = _lookup_flat(tok.reshape(-1), embed_weights)
    return out.reshape(tokens.shape + (d_model,))
